# baseline (speedup 1.0000x reference)
import functools

import jax
import jax.numpy as jnp
from jax import lax
from jax.experimental import pallas as pl
from jax.experimental.pallas import tpu as pltpu
from jax.experimental.pallas import tpu_sc as plsc

N = 10000
E = 320000
FEATS = 128
CLASSES = 64
DEPTH = 10

NC = 2
NS = 16
HALF = FEATS // NC
NPAD = 10240
RPT = NPAD // NS
RCH = RPT // 128
EPAD = 327680
EPT = EPAD // NS
ECH = EPT // 128
EGRP = 32
NGRP = ECH // EGRP


def _sc_body(xh, srcE, dstE, prm, Bt, xout, z_hbm,
             u_sp, deg_sp,
             srcg, dstg, big0, big1, big2,
             degt, dinv_t, drt_t, c1_t, s_t,
             onesb, prmv, Btv, sem0, sem1, sem2, sem3, sem4, sem5):
    bufs = (big0.at[pl.ds(0, 128)], big0.at[pl.ds(128, 128)],
            big1.at[pl.ds(0, 128)], big1.at[pl.ds(128, 128)],
            big2)
    sems = (sem0, sem1, sem2, sem3, sem4, sem5)
    bufA, bufB, bufC = bufs[0], bufs[1], bufs[2]
    c = lax.axis_index("c")
    sid = lax.axis_index("s")
    r0 = sid * RPT
    zref = z_hbm.at[c]

    pltpu.sync_copy(prm, prmv)
    pltpu.sync_copy(Bt.at[c], Btv)
    pltpu.sync_copy(srcE.at[sid], srcg)
    pltpu.sync_copy(dstE.at[sid], dstg)

    @pl.loop(0, 8)
    def _(i):
        onesb[pl.ds(i * 16, 16)] = jnp.full((16,), 1.0, jnp.float32)

    @pl.loop(0, RPT // 16)
    def _(i):
        degt[pl.ds(i * 16, 16)] = jnp.zeros((16,), jnp.float32)

    pltpu.sync_copy(degt, deg_sp.at[pl.ds(r0, RPT)])
    plsc.subcore_barrier()

    @pl.loop(0, ECH)
    def _(j):
        pltpu.sync_copy(onesb, deg_sp.at[dstg.at[j]], add=True)
    plsc.subcore_barrier()

    pltpu.sync_copy(deg_sp.at[pl.ds(r0, RPT)], degt)

    @pl.loop(0, RPT // 16)
    def _(i):
        sl = pl.ds(i * 16, 16)
        d = jnp.maximum(degt[sl], 1.0)
        ii = lax.bitcast_convert_type(d, jnp.int32)
        y = lax.bitcast_convert_type(jnp.int32(0x5F3759DF) - (ii >> 1), jnp.float32)
        y = y * (1.5 - 0.5 * d * y * y)
        y = y * (1.5 - 0.5 * d * y * y)
        y = y * (1.5 - 0.5 * d * y * y)
        dinv_t[sl] = y
        drt_t[sl] = d * y
        c1_t[sl] = 0.9 * y * y
        s_t[sl] = d * y * (1.0 / 9.0)

    def rowpass(srefs, base, body_fn):
        @pl.loop(0, 8)
        def _(i):
            vecs = [sr[pl.ds(base + i * 16, 16)] for sr in srefs]
            for j in range(16):
                body_fn(i * 16 + j, *[v[j] for v in vecs])

    def stack_setup(vsrc):
        @pl.loop(0, RCH)
        def _(k):
            rows = pl.ds(r0 + k * 128, 128)
            pltpu.sync_copy(vsrc.at[rows], bufA)

            def setup_row(r, di, sv):
                for f in range(4):
                    fs = pl.ds(f * 16, 16)
                    xv = bufA[r, fs]
                    bufB[r, fs] = xv * di
                    bufC[r, fs] = xv * sv

            rowpass([dinv_t, s_t], k * 128, setup_row)
            pltpu.sync_copy(bufB, zref.at[rows])
            pltpu.sync_copy(bufC, u_sp.at[rows])


    def edge_phase():
        nd = 5
        for b in range(nd):
            pltpu.async_copy(zref.at[srcg.at[b]], bufs[b], sems[b])

        @pl.loop(0, ECH, step=nd)
        def _(jj):
            for b in range(nd):
                pltpu.make_async_copy(
                    zref.at[srcg.at[jj + b]], bufs[b], sems[b]).wait()
                pltpu.sync_copy(bufs[b], u_sp.at[dstg.at[jj + b]], add=True)

                @pl.when(jj + nd + b < ECH)
                def _(b=b):
                    pltpu.async_copy(
                        zref.at[srcg.at[jj + nd + b]], bufs[b], sems[b])

    def rowsk(k):
        return pl.ds(r0 + k * 128, 128)

    def z_update(vsrc):
        pltpu.async_copy(u_sp.at[rowsk(0)], bufs[0], sems[0])
        pltpu.async_copy(vsrc.at[rowsk(0)], bufs[1], sems[1])
        for k in range(RCH):
            pa, sa = bufs[2 * (k % 2)], sems[2 * (k % 2)]
            pb, sb = bufs[2 * (k % 2) + 1], sems[2 * (k % 2) + 1]
            na, nsa = bufs[2 * ((k + 1) % 2)], sems[2 * ((k + 1) % 2)]
            nb, nsb = bufs[2 * ((k + 1) % 2) + 1], sems[2 * ((k + 1) % 2) + 1]
            if k + 1 < RCH:
                if k >= 1:
                    pltpu.make_async_copy(na, zref.at[rowsk(k - 1)], nsa).wait()
                    pltpu.make_async_copy(nb, u_sp.at[rowsk(k - 1)], nsb).wait()
                pltpu.async_copy(u_sp.at[rowsk(k + 1)], na, nsa)
                pltpu.async_copy(vsrc.at[rowsk(k + 1)], nb, nsb)
            pltpu.make_async_copy(u_sp.at[rowsk(k)], pa, sa).wait()
            pltpu.make_async_copy(vsrc.at[rowsk(k)], pb, sb).wait()

            def zrow(r, cv, sv, pa=pa, pb=pb):
                for f in range(4):
                    fs = pl.ds(f * 16, 16)
                    pa[r, fs] = pa[r, fs] * cv
                    pb[r, fs] = pb[r, fs] * sv

            rowpass([c1_t, s_t], k * 128, zrow)
            pltpu.async_copy(pa, zref.at[rowsk(k)], sa)
            pltpu.async_copy(pb, u_sp.at[rowsk(k)], sb)
        for k in (RCH - 2, RCH - 1):
            pltpu.make_async_copy(
                bufs[2 * (k % 2)], zref.at[rowsk(k)], sems[2 * (k % 2)]).wait()
            pltpu.make_async_copy(
                bufs[2 * (k % 2) + 1], u_sp.at[rowsk(k)],
                sems[2 * (k % 2) + 1]).wait()

    def diffusion_stack(vsrc):
        @pl.loop(0, DEPTH)
        def _(t):
            edge_phase()
            plsc.subcore_barrier()
            z_update(vsrc)
            plsc.subcore_barrier()

    stack_setup(xh.at[c])
    plsc.subcore_barrier()
    diffusion_stack(xh.at[c])

    av = prmv[0]
    wv = prmv[1]
    b2 = prmv[2][0]

    @pl.loop(0, RCH)
    def _(k):
        rows = pl.ds(r0 + k * 128, 128)
        pltpu.sync_copy(zref.at[rows], bufA)

        def pre_row(r, dr):
            for f in range(4):
                fs = pl.ds(f * 16, 16)
                bufA[r, fs] = bufA[r, fs] * dr

        rowpass([drt_t], k * 128, pre_row)

        for f in range(4):
            fs = pl.ds(f * 16, 16)
            bvs = [Btv[h, fs] for h in range(11)]

            @pl.loop(0, 128)
            def _(r, fs=fs, bvs=bvs):
                xv = bufA[r, fs]
                acc = jnp.full((16,), b2, jnp.float32)
                for h in range(11):
                    hv = jnp.maximum(xv * av[h] + bvs[h], 0.0)
                    acc = acc + hv * wv[h]
                bufB[r, fs] = acc

        def post_row(r, di, sv):
            for f in range(4):
                fs = pl.ds(f * 16, 16)
                yv = bufB[r, fs]
                bufA[r, fs] = yv * di
                bufC[r, fs] = yv * sv

        rowpass([dinv_t, s_t], k * 128, post_row)
        pltpu.sync_copy(bufB, xout.at[c, rows])
        pltpu.sync_copy(bufA, zref.at[rows])
        pltpu.sync_copy(bufC, u_sp.at[rows])
    plsc.subcore_barrier()

    diffusion_stack(xout.at[c])

    @pl.loop(0, RCH)
    def _(k):
        rows = pl.ds(r0 + k * 128, 128)
        pltpu.sync_copy(zref.at[rows], bufA)

        def fin_row(r, dr):
            for f in range(4):
                fs = pl.ds(f * 16, 16)
                bufA[r, fs] = bufA[r, fs] * dr

        rowpass([drt_t], k * 128, fin_row)
        pltpu.sync_copy(bufA, xout.at[c, rows])


@functools.cache
def _sc_diffusion():
  return pl.kernel(
    _sc_body,
    out_type=(
        jax.ShapeDtypeStruct((NC, NPAD, HALF), jnp.float32),
        jax.ShapeDtypeStruct((NC, NPAD, HALF), jnp.float32),
    ),
    mesh=plsc.VectorSubcoreMesh(
        core_axis_name="c", subcore_axis_name="s", num_cores=NC, num_subcores=NS
    ),
    compiler_params=pltpu.CompilerParams(use_tc_tiling_on_sc=False),
    scratch_types=[
        pltpu.VMEM_SHARED((NPAD, HALF), jnp.float32),
        pltpu.VMEM_SHARED((NPAD,), jnp.float32),
        pltpu.VMEM((ECH, 128), jnp.int32),
        pltpu.VMEM((ECH, 128), jnp.int32),
        pltpu.VMEM((256, HALF), jnp.float32),
        pltpu.VMEM((256, HALF), jnp.float32),
        pltpu.VMEM((128, HALF), jnp.float32),
        pltpu.VMEM((RPT,), jnp.float32),
        pltpu.VMEM((RPT,), jnp.float32),
        pltpu.VMEM((RPT,), jnp.float32),
        pltpu.VMEM((RPT,), jnp.float32),
        pltpu.VMEM((RPT,), jnp.float32),
        pltpu.VMEM((128,), jnp.float32),
        pltpu.VMEM((3, 16), jnp.float32),
        pltpu.VMEM((11, HALF), jnp.float32),
        pltpu.SemaphoreType.DMA,
        pltpu.SemaphoreType.DMA,
        pltpu.SemaphoreType.DMA,
        pltpu.SemaphoreType.DMA,
        pltpu.SemaphoreType.DMA,
        pltpu.SemaphoreType.DMA,
    ],
  )


def _mm_body(xa, xb, wa, wb, b, o):
    o[...] = (
        jnp.dot(xa[0], wa[...], preferred_element_type=jnp.float32)
        + jnp.dot(xb[0], wb[...], preferred_element_type=jnp.float32)
        + b[...]
    )


def kernel(x, edges, emb, A1_w, A1_b, A2_w, A2_b, W1, b1):
    f32 = jnp.float32
    xp = jnp.pad(x, ((0, NPAD - N), (0, 0)))
    xh = jnp.stack([xp[:, :HALF], xp[:, HALF:]])

    padidx = (jnp.arange(EPAD - E, dtype=jnp.int32) % (NPAD - N)) + N
    srcE = jnp.concatenate([edges[0], padidx]).reshape(NS, ECH, 128)
    dstE = jnp.concatenate([edges[1], padidx]).reshape(NS, ECH, 128)

    a1 = A1_w[0]
    Bm = emb @ A1_w[1:] + A1_b
    Bt = jnp.stack([Bm[:HALF].T, Bm[HALF:].T])
    prm = (
        jnp.zeros((3, 16), f32)
        .at[0, :11].set(a1)
        .at[1, :11].set(A2_w[:, 0])
        .at[2, 0].set(A2_b[0])
    )

    xout, _zscratch = _sc_diffusion()(xh, srcE, dstE, prm, Bt)

    blk = NPAD // 8
    out = pl.pallas_call(
        _mm_body,
        grid=(8,),
        in_specs=[
            pl.BlockSpec((1, blk, HALF), lambda i: (0, i, 0)),
            pl.BlockSpec((1, blk, HALF), lambda i: (1, i, 0)),
            pl.BlockSpec((HALF, CLASSES), lambda i: (0, 0)),
            pl.BlockSpec((HALF, CLASSES), lambda i: (0, 0)),
            pl.BlockSpec((1, CLASSES), lambda i: (0, 0)),
        ],
        out_specs=pl.BlockSpec((blk, CLASSES), lambda i: (i, 0)),
        out_shape=jax.ShapeDtypeStruct((NPAD, CLASSES), f32),
    )(xout, xout, W1[:HALF], W1[HALF:], b1[None, :])
    return out[:N]


# speedup vs baseline: 1.0041x; 1.0041x over previous
import functools

import jax
import jax.numpy as jnp
from jax import lax
from jax.experimental import pallas as pl
from jax.experimental.pallas import tpu as pltpu
from jax.experimental.pallas import tpu_sc as plsc

N = 10000
E = 320000
FEATS = 128
CLASSES = 64
DEPTH = 10

NC = 2
NS = 16
HALF = FEATS // NC
NPAD = 10240
RPT = NPAD // NS
RCH = RPT // 128
EPAD = 327680
EPT = EPAD // NS
ECH = EPT // 128
EGRP = 32
NGRP = ECH // EGRP


def _sc_body(xh, srcE, dstE, prm, Bt, xout, z_hbm, drt_hbm,
             u_sp, deg_sp,
             srcg, dstg, big0, big1,
             degt, dinv_t, drt_t, c1_t, s_t,
             onesb, prmv, Btv, sem0, sem1, sem2, sem3):
    bufs = (big0.at[pl.ds(0, 128)], big0.at[pl.ds(128, 128)],
            big1.at[pl.ds(0, 128)], big1.at[pl.ds(128, 128)])
    sems = (sem0, sem1, sem2, sem3)
    bufA, bufB, bufC = bufs[0], bufs[1], bufs[2]
    c = lax.axis_index("c")
    sid = lax.axis_index("s")
    r0 = sid * RPT
    zref = z_hbm.at[c]

    pltpu.sync_copy(prm, prmv)
    pltpu.sync_copy(Bt.at[c], Btv)
    pltpu.sync_copy(srcE.at[sid], srcg)
    pltpu.sync_copy(dstE.at[sid], dstg)

    @pl.loop(0, 8)
    def _(i):
        onesb[pl.ds(i * 16, 16)] = jnp.full((16,), 1.0, jnp.float32)

    @pl.loop(0, RPT // 16)
    def _(i):
        degt[pl.ds(i * 16, 16)] = jnp.zeros((16,), jnp.float32)

    pltpu.sync_copy(degt, deg_sp.at[pl.ds(r0, RPT)])
    plsc.subcore_barrier()

    @pl.loop(0, ECH)
    def _(j):
        pltpu.sync_copy(onesb, deg_sp.at[dstg.at[j]], add=True)
    plsc.subcore_barrier()

    pltpu.sync_copy(deg_sp.at[pl.ds(r0, RPT)], degt)

    @pl.loop(0, RPT // 16)
    def _(i):
        sl = pl.ds(i * 16, 16)
        d = jnp.maximum(degt[sl], 1.0)
        ii = lax.bitcast_convert_type(d, jnp.int32)
        y = lax.bitcast_convert_type(jnp.int32(0x5F3759DF) - (ii >> 1), jnp.float32)
        y = y * (1.5 - 0.5 * d * y * y)
        y = y * (1.5 - 0.5 * d * y * y)
        y = y * (1.5 - 0.5 * d * y * y)
        dinv_t[sl] = y
        drt_t[sl] = d * y
        c1_t[sl] = 0.9 * y * y
        s_t[sl] = d * y * (1.0 / 9.0)

    @pl.when(c == 0)
    def _():
        pltpu.sync_copy(drt_t, drt_hbm.at[pl.ds(r0, RPT)])

    def rowpass(srefs, base, body_fn):
        @pl.loop(0, 8)
        def _(i):
            vecs = [sr[pl.ds(base + i * 16, 16)] for sr in srefs]
            for j in range(16):
                body_fn(i * 16 + j, *[v[j] for v in vecs])

    def stack_setup(vsrc):
        @pl.loop(0, RCH)
        def _(k):
            rows = pl.ds(r0 + k * 128, 128)
            pltpu.sync_copy(vsrc.at[rows], bufA)

            def setup_row(r, di, sv):
                for f in range(4):
                    fs = pl.ds(f * 16, 16)
                    xv = bufA[r, fs]
                    bufB[r, fs] = xv * di
                    bufC[r, fs] = xv * sv

            rowpass([dinv_t, s_t], k * 128, setup_row)
            pltpu.sync_copy(bufB, zref.at[rows])
            pltpu.sync_copy(bufC, u_sp.at[rows])


    def edge_phase():
        for b in range(4):
            pltpu.async_copy(zref.at[srcg.at[b]], bufs[b], sems[b])

        @pl.loop(0, ECH, step=4)
        def _(jj):
            for b in range(4):
                pltpu.make_async_copy(
                    zref.at[srcg.at[jj + b]], bufs[b], sems[b]).wait()
                pltpu.sync_copy(bufs[b], u_sp.at[dstg.at[jj + b]], add=True)

                @pl.when(jj + 4 + b < ECH)
                def _(b=b):
                    pltpu.async_copy(
                        zref.at[srcg.at[jj + 4 + b]], bufs[b], sems[b])

    def rowsk(k):
        return pl.ds(r0 + k * 128, 128)

    def z_update(vsrc):
        pltpu.async_copy(u_sp.at[rowsk(0)], bufs[0], sems[0])
        pltpu.async_copy(vsrc.at[rowsk(0)], bufs[1], sems[1])
        for k in range(RCH):
            pa, sa = bufs[2 * (k % 2)], sems[2 * (k % 2)]
            pb, sb = bufs[2 * (k % 2) + 1], sems[2 * (k % 2) + 1]
            na, nsa = bufs[2 * ((k + 1) % 2)], sems[2 * ((k + 1) % 2)]
            nb, nsb = bufs[2 * ((k + 1) % 2) + 1], sems[2 * ((k + 1) % 2) + 1]
            if k + 1 < RCH:
                if k >= 1:
                    pltpu.make_async_copy(na, zref.at[rowsk(k - 1)], nsa).wait()
                    pltpu.make_async_copy(nb, u_sp.at[rowsk(k - 1)], nsb).wait()
                pltpu.async_copy(u_sp.at[rowsk(k + 1)], na, nsa)
                pltpu.async_copy(vsrc.at[rowsk(k + 1)], nb, nsb)
            pltpu.make_async_copy(u_sp.at[rowsk(k)], pa, sa).wait()
            pltpu.make_async_copy(vsrc.at[rowsk(k)], pb, sb).wait()

            def zrow(r, cv, sv, pa=pa, pb=pb):
                for f in range(4):
                    fs = pl.ds(f * 16, 16)
                    pa[r, fs] = pa[r, fs] * cv
                    pb[r, fs] = pb[r, fs] * sv

            rowpass([c1_t, s_t], k * 128, zrow)
            pltpu.async_copy(pa, zref.at[rowsk(k)], sa)
            pltpu.async_copy(pb, u_sp.at[rowsk(k)], sb)
        for k in (RCH - 2, RCH - 1):
            pltpu.make_async_copy(
                bufs[2 * (k % 2)], zref.at[rowsk(k)], sems[2 * (k % 2)]).wait()
            pltpu.make_async_copy(
                bufs[2 * (k % 2) + 1], u_sp.at[rowsk(k)],
                sems[2 * (k % 2) + 1]).wait()

    def diffusion_stack(vsrc):
        @pl.loop(0, DEPTH)
        def _(t):
            edge_phase()
            plsc.subcore_barrier()
            z_update(vsrc)
            plsc.subcore_barrier()

    stack_setup(xh.at[c])
    plsc.subcore_barrier()
    diffusion_stack(xh.at[c])

    av = prmv[0]
    wv = prmv[1]
    b2 = prmv[2][0]

    @pl.loop(0, RCH)
    def _(k):
        rows = pl.ds(r0 + k * 128, 128)
        pltpu.sync_copy(zref.at[rows], bufA)

        def pre_row(r, dr):
            for f in range(4):
                fs = pl.ds(f * 16, 16)
                bufA[r, fs] = bufA[r, fs] * dr

        rowpass([drt_t], k * 128, pre_row)

        for f in range(4):
            fs = pl.ds(f * 16, 16)
            bvs = [Btv[h, fs] for h in range(11)]

            @pl.loop(0, 128)
            def _(r, fs=fs, bvs=bvs):
                xv = bufA[r, fs]
                acc = jnp.full((16,), b2, jnp.float32)
                for h in range(11):
                    hv = jnp.maximum(xv * av[h] + bvs[h], 0.0)
                    acc = acc + hv * wv[h]
                bufB[r, fs] = acc

        def post_row(r, di, sv):
            for f in range(4):
                fs = pl.ds(f * 16, 16)
                yv = bufB[r, fs]
                bufA[r, fs] = yv * di
                bufC[r, fs] = yv * sv

        rowpass([dinv_t, s_t], k * 128, post_row)
        pltpu.sync_copy(bufB, xout.at[c, rows])
        pltpu.sync_copy(bufA, zref.at[rows])
        pltpu.sync_copy(bufC, u_sp.at[rows])
    plsc.subcore_barrier()

    diffusion_stack(xout.at[c])


@functools.cache
def _sc_diffusion():
  return pl.kernel(
    _sc_body,
    out_type=(
        jax.ShapeDtypeStruct((NC, NPAD, HALF), jnp.float32),
        jax.ShapeDtypeStruct((NC, NPAD, HALF), jnp.float32),
        jax.ShapeDtypeStruct((NPAD,), jnp.float32),
    ),
    mesh=plsc.VectorSubcoreMesh(
        core_axis_name="c", subcore_axis_name="s", num_cores=NC, num_subcores=NS
    ),
    compiler_params=pltpu.CompilerParams(use_tc_tiling_on_sc=False),
    scratch_types=[
        pltpu.VMEM_SHARED((NPAD, HALF), jnp.float32),
        pltpu.VMEM_SHARED((NPAD,), jnp.float32),
        pltpu.VMEM((ECH, 128), jnp.int32),
        pltpu.VMEM((ECH, 128), jnp.int32),
        pltpu.VMEM((256, HALF), jnp.float32),
        pltpu.VMEM((256, HALF), jnp.float32),
        pltpu.VMEM((RPT,), jnp.float32),
        pltpu.VMEM((RPT,), jnp.float32),
        pltpu.VMEM((RPT,), jnp.float32),
        pltpu.VMEM((RPT,), jnp.float32),
        pltpu.VMEM((RPT,), jnp.float32),
        pltpu.VMEM((128,), jnp.float32),
        pltpu.VMEM((3, 16), jnp.float32),
        pltpu.VMEM((11, HALF), jnp.float32),
        pltpu.SemaphoreType.DMA,
        pltpu.SemaphoreType.DMA,
        pltpu.SemaphoreType.DMA,
        pltpu.SemaphoreType.DMA,
    ],
  )


def _mm_body(za, zb, d, wa, wb, b, o):
    o[...] = (
        jnp.dot(za[0] * d[...], wa[...], preferred_element_type=jnp.float32)
        + jnp.dot(zb[0] * d[...], wb[...], preferred_element_type=jnp.float32)
        + b[...]
    )


def kernel(x, edges, emb, A1_w, A1_b, A2_w, A2_b, W1, b1):
    f32 = jnp.float32
    xp = jnp.pad(x, ((0, NPAD - N), (0, 0)))
    xh = jnp.stack([xp[:, :HALF], xp[:, HALF:]])

    padidx = (jnp.arange(EPAD - E, dtype=jnp.int32) % (NPAD - N)) + N
    srcE = jnp.concatenate([edges[0], padidx]).reshape(NS, ECH, 128)
    dstE = jnp.concatenate([edges[1], padidx]).reshape(NS, ECH, 128)

    a1 = A1_w[0]
    Bm = emb @ A1_w[1:] + A1_b
    Bt = jnp.stack([Bm[:HALF].T, Bm[HALF:].T])
    prm = (
        jnp.zeros((3, 16), f32)
        .at[0, :11].set(a1)
        .at[1, :11].set(A2_w[:, 0])
        .at[2, 0].set(A2_b[0])
    )

    _y, zfin, drt = _sc_diffusion()(xh, srcE, dstE, prm, Bt)

    blk = NPAD // 8
    out = pl.pallas_call(
        _mm_body,
        grid=(8,),
        in_specs=[
            pl.BlockSpec((1, blk, HALF), lambda i: (0, i, 0)),
            pl.BlockSpec((1, blk, HALF), lambda i: (1, i, 0)),
            pl.BlockSpec((blk, 1), lambda i: (i, 0)),
            pl.BlockSpec((HALF, CLASSES), lambda i: (0, 0)),
            pl.BlockSpec((HALF, CLASSES), lambda i: (0, 0)),
            pl.BlockSpec((1, CLASSES), lambda i: (0, 0)),
        ],
        out_specs=pl.BlockSpec((blk, CLASSES), lambda i: (i, 0)),
        out_shape=jax.ShapeDtypeStruct((NPAD, CLASSES), f32),
    )(zfin, zfin, drt[:, None], W1[:HALF], W1[HALF:], b1[None, :])
    return out[:N]
